# baseline (speedup 1.0000x reference)
import functools

import jax
import jax.numpy as jnp
from jax import lax
from jax.experimental import pallas as pl
from jax.experimental.pallas import tpu as pltpu
from jax.experimental.pallas import tpu_sc as plsc

_N = 50000
_E = 800000
_C = 16
_K = 128
_H = 64
_NCHUNK = _E // _K
_MAIN = 390
_EXTRA = _NCHUNK - 16 * _MAIN
_ACC_ROWS = 50048
_RPT = _ACC_ROWS // 16

_mesh = plsc.VectorSubcoreMesh(core_axis_name="c", subcore_axis_name="s")


@functools.partial(
    pl.kernel,
    out_type=jax.ShapeDtypeStruct((2 * _ACC_ROWS, 32), jnp.float32),
    mesh=_mesh,
    compiler_params=pltpu.CompilerParams(needs_layout_passes=False,
                                         use_tc_tiling_on_sc=False),
    scratch_types=[
        [pltpu.VMEM((_K,), jnp.int32)] * 2,
        [pltpu.VMEM((2, _H), jnp.int32)] * 2,
        [pltpu.VMEM((2 * 1024,), jnp.float32)] * 2,
        [pltpu.VMEM((6 * 1024,), jnp.float32)] * 2,
        [pltpu.VMEM((_H, 64), jnp.float32)] * 2,
        [pltpu.VMEM((_H, 32), jnp.float32)] * 2,
        [pltpu.VMEM((_H,), jnp.int32)] * 2,
        pltpu.VMEM_SHARED((_ACC_ROWS, 32), jnp.float32),
        [pltpu.SemaphoreType.DMA] * 2,
        [pltpu.SemaphoreType.DMA] * 2,
        [pltpu.SemaphoreType.DMA] * 2,
        [pltpu.SemaphoreType.DMA] * 2,
    ],
)
def _edge_kernel(htab, srcm, nbrm, ea0m, ea1m, out_hbm,
                 nbr, src, ea0, ea1, g, o, idx, acc,
                 semidx, semea, semg, semsc):
    c = lax.axis_index("c")
    s = lax.axis_index("s")
    nt = jnp.where(s < _EXTRA, _MAIN + 1, _MAIN)

    cm = (jnp.zeros((_C,), jnp.int32) + c) == 0
    zeros16 = jnp.zeros((_C,), jnp.float32)
    iota = lax.iota(jnp.int32, _C)
    chb = iota // 8
    ccol = chb * 1024 + (iota - chb * 8) * 128
    a1c = [ccol, ccol + 2048, ccol + 4096]

    def _idx_refs(j, r):
        return (
            (nbrm.at[pl.ds(j * _K, _K)], nbr[r].at[pl.ds(0, _K)]),
            (srcm.at[pl.ds(j * _K, _H)], src[r].at[0]),
            (srcm.at[pl.ds(j * _K + _H, _H)], src[r].at[1]),
        )

    def _ea_refs(j, r):
        refs = []
        for q in range(2):
            refs.append((ea0m.at[pl.ds((q * _NCHUNK + j) * 1024, 1024)],
                         ea0[r].at[pl.ds(q * 1024, 1024)]))
        for q in range(6):
            refs.append((ea1m.at[pl.ds((q * _NCHUNK + j) * 1024, 1024)],
                         ea1[r].at[pl.ds(q * 1024, 1024)]))
        return refs

    def _issue_lin(j, r):
        for a, b in _idx_refs(j, r):
            pltpu.async_copy(a, b, semidx[r])
        for a, b in _ea_refs(j, r):
            pltpu.async_copy(a, b, semea[r])

    def _wait_idx(j, r):
        for a, b in _idx_refs(j, r):
            pltpu.make_async_copy(a, b, semidx[r]).wait()

    def _wait_ea(j, r):
        for a, b in _ea_refs(j, r):
            pltpu.make_async_copy(a, b, semea[r]).wait()

    def _gather(r, h, p):
        pltpu.async_copy(htab.at[nbr[r].at[pl.ds(h * _H, _H)]], g[p], semg[p])

    def _wait_gather(r, h, p):
        pltpu.make_async_copy(htab.at[nbr[r].at[pl.ds(h * _H, _H)]],
                              g[p], semg[p]).wait()

    def _wait_scatter(h):
        pltpu.make_async_copy(o[h], acc.at[idx[h]], semsc[h]).wait()

    _issue_lin(s, 0)

    @plsc.parallel_loop(0, _H)
    def _zero_row(e):
        o[0][e, pl.ds(0, 16)] = zeros16
        o[0][e, pl.ds(16, 16)] = zeros16

    row0 = s * _RPT
    for k in range(_RPT // _H):
        pltpu.sync_copy(o[0], acc.at[pl.ds(row0 + k * _H, _H)])
    pltpu.sync_copy(o[0].at[pl.ds(0, _RPT % _H)],
                    acc.at[pl.ds(row0 + _RPT // _H * _H, _RPT % _H)])

    _wait_idx(s, 0)
    _gather(0, 0, 0)
    plsc.subcore_barrier()

    def body(u, h, r):
        n = u // 2
        r1 = 1 - r
        j1 = s + (n + 1) * 16

        if h == 0:
            @pl.when(n + 1 < nt)
            def _():
                _issue_lin(j1, r1)
            _gather(r, 1, 1)
            _wait_gather(r, 0, 0)
        else:
            @pl.when(n + 1 < nt)
            def _():
                _wait_idx(j1, r1)
                _gather(r1, 0, 0)
            _wait_gather(r, 1, 1)

        @pl.when(u >= 2)
        def _():
            _wait_scatter(h)

        if h == 0:
            _wait_ea(s + n * 16, r)

        @plsc.parallel_loop(0, _H // _C)
        def _idx_copy(i):
            idx[h][pl.ds(i * _C, _C)] = src[r][h, pl.ds(i * _C, _C)]

        gv, e0v, e1v, ov = g[h], ea0[r], ea1[r], o[h]

        @plsc.parallel_loop(0, _H, unroll=2)
        def _edge(e):
            g0 = gv[e, pl.ds(0, 16)]
            g1x = gv[e, pl.ds(16, 16)]
            g1y = gv[e, pl.ds(32, 16)]
            g1z = gv[e, pl.ds(48, 16)]
            es = jnp.full((_C,), e + h * _H, jnp.int32)
            a0 = plsc.load_gather(e0v, [ccol + es])
            a1x = plsc.load_gather(e1v, [a1c[0] + es])
            a1y = plsc.load_gather(e1v, [a1c[1] + es])
            a1z = plsc.load_gather(e1v, [a1c[2] + es])
            o0 = g0 * a0 + g1x * a1x + g1y * a1y + g1z * a1z
            px = g0 * a1x + g1x * a0
            py = g0 * a1y + g1y * a0
            pz = g0 * a1z + g1z * a0
            ov[e, pl.ds(0, 16)] = jnp.where(cm, o0, py)
            ov[e, pl.ds(16, 16)] = jnp.where(cm, px, pz)

        pltpu.async_copy(ov, acc.at[idx[h]], semsc[h], add=True)

    def outer(m, _):
        for k in range(4):
            u = m * 4 + k
            body(u, k % 2, (k // 2) % 2)
        return 0

    lax.fori_loop(0, 2 * _MAIN // 4, outer, 0)

    @pl.when(s < _EXTRA)
    def _():
        body(2 * _MAIN, 0, 0)
        body(2 * _MAIN + 1, 1, 0)

    _wait_scatter(0)
    _wait_scatter(1)
    plsc.subcore_barrier()

    pltpu.sync_copy(acc.at[pl.ds(row0, _RPT)],
                    out_hbm.at[pl.ds(c * _ACC_ROWS + row0, _RPT)])


def kernel(h_0, h_1, rel_pos, edge_index, edge_attr_0, edge_attr_1,
           channel_weights):
    del rel_pos, channel_weights
    n = h_0.shape[0]
    nb = _NCHUNK
    htab = jnp.concatenate(
        [h_0, h_1[:, :, 0], h_1[:, :, 1], h_1[:, :, 2],
         jnp.zeros((n, 64), jnp.float32)], axis=1).reshape(2 * n, 64)
    src = edge_index[0].astype(jnp.int32)
    nbr2 = edge_index[1].astype(jnp.int32) * 2
    ea0_t = (edge_attr_0.T.reshape(2, 8, nb, 128)
             .transpose(0, 2, 1, 3).reshape(-1))
    ea1_t = (edge_attr_1.transpose(2, 1, 0).reshape(3, 2, 8, nb, 128)
             .transpose(0, 1, 3, 2, 4).reshape(-1))
    out = _edge_kernel(htab, src, nbr2, ea0_t,
                       ea1_t).reshape(2, _ACC_ROWS, 32)
    a, b = out[0, :n], out[1, :n]
    out0 = a[:, :16]
    out1 = jnp.stack([a[:, 16:], b[:, :16], b[:, 16:]], axis=-1)
    return (out0, out1)


# speedup vs baseline: 1.0121x; 1.0121x over previous
import functools

import jax
import jax.numpy as jnp
from jax import lax
from jax.experimental import pallas as pl
from jax.experimental.pallas import tpu as pltpu
from jax.experimental.pallas import tpu_sc as plsc

_N = 50000
_E = 800000
_C = 16
_K = 128
_H = 64
_NCHUNK = _E // _K
_MAIN = 390
_EXTRA = _NCHUNK - 16 * _MAIN
_ACC_ROWS = 50048
_RPT = _ACC_ROWS // 16

_mesh = plsc.VectorSubcoreMesh(core_axis_name="c", subcore_axis_name="s")


@functools.partial(
    pl.kernel,
    out_type=jax.ShapeDtypeStruct((2 * _ACC_ROWS, 32), jnp.float32),
    mesh=_mesh,
    compiler_params=pltpu.CompilerParams(needs_layout_passes=False,
                                         use_tc_tiling_on_sc=False),
    scratch_types=[
        [pltpu.VMEM((_K,), jnp.int32)] * 2,
        [pltpu.VMEM((2, _H), jnp.int32)] * 2,
        [pltpu.VMEM((2 * 1024,), jnp.float32)] * 2,
        [pltpu.VMEM((6 * 1024,), jnp.float32)] * 2,
        [pltpu.VMEM((_H, 64), jnp.float32)] * 2,
        [pltpu.VMEM((_H, 32), jnp.float32)] * 2,
        [pltpu.VMEM((_H,), jnp.int32)] * 2,
        pltpu.VMEM_SHARED((_ACC_ROWS, 32), jnp.float32),
        [pltpu.SemaphoreType.DMA] * 2,
        [pltpu.SemaphoreType.DMA] * 2,
        [pltpu.SemaphoreType.DMA] * 2,
        [pltpu.SemaphoreType.DMA] * 2,
    ],
)
def _edge_kernel(htab, srcm, nbrm, ea0m, ea1m, out_hbm,
                 nbr, src, ea0, ea1, g, o, idx, acc,
                 semidx, semea, semg, semsc):
    c = lax.axis_index("c")
    s = lax.axis_index("s")
    nt = jnp.where(s < _EXTRA, _MAIN + 1, _MAIN)

    cm = (jnp.zeros((_C,), jnp.int32) + c) == 0
    zeros16 = jnp.zeros((_C,), jnp.float32)
    iota = lax.iota(jnp.int32, _C)
    chb = iota // 8
    ccol = chb * 1024 + (iota - chb * 8) * 128
    a1c = [ccol, ccol + 2048, ccol + 4096]

    def _idx_refs(j, r):
        return (
            (nbrm.at[pl.ds(j * _K, _K)], nbr[r].at[pl.ds(0, _K)]),
            (srcm.at[pl.ds(j * _K, _H)], src[r].at[0]),
            (srcm.at[pl.ds(j * _K + _H, _H)], src[r].at[1]),
        )

    def _ea_refs(j, r):
        refs = []
        for q in range(2):
            refs.append((ea0m.at[pl.ds((q * _NCHUNK + j) * 1024, 1024)],
                         ea0[r].at[pl.ds(q * 1024, 1024)]))
        for q in range(6):
            refs.append((ea1m.at[pl.ds((q * _NCHUNK + j) * 1024, 1024)],
                         ea1[r].at[pl.ds(q * 1024, 1024)]))
        return refs

    def _issue_lin(j, r):
        for a, b in _idx_refs(j, r):
            pltpu.async_copy(a, b, semidx[r])
        for a, b in _ea_refs(j, r):
            pltpu.async_copy(a, b, semea[r])

    def _wait_idx(j, r):
        for a, b in _idx_refs(j, r):
            pltpu.make_async_copy(a, b, semidx[r]).wait()

    def _wait_ea(j, r):
        for a, b in _ea_refs(j, r):
            pltpu.make_async_copy(a, b, semea[r]).wait()

    def _gather(r, h, p):
        pltpu.async_copy(htab.at[nbr[r].at[pl.ds(h * _H, _H)]], g[p], semg[p])

    def _wait_gather(r, h, p):
        pltpu.make_async_copy(htab.at[nbr[r].at[pl.ds(h * _H, _H)]],
                              g[p], semg[p]).wait()

    def _wait_scatter(h):
        pltpu.make_async_copy(o[h], acc.at[idx[h]], semsc[h]).wait()

    _issue_lin(s, 0)

    @plsc.parallel_loop(0, _H)
    def _zero_row(e):
        o[0][e, pl.ds(0, 16)] = zeros16
        o[0][e, pl.ds(16, 16)] = zeros16

    row0 = s * _RPT
    for k in range(_RPT // _H):
        pltpu.sync_copy(o[0], acc.at[pl.ds(row0 + k * _H, _H)])
    pltpu.sync_copy(o[0].at[pl.ds(0, _RPT % _H)],
                    acc.at[pl.ds(row0 + _RPT // _H * _H, _RPT % _H)])

    _wait_idx(s, 0)
    _gather(0, 0, 0)
    plsc.subcore_barrier()

    def body(u, h, r):
        n = u // 2
        r1 = 1 - r
        j1 = s + (n + 1) * 16

        if h == 0:
            @pl.when(n + 1 < nt)
            def _():
                _issue_lin(j1, r1)
            _gather(r, 1, 1)
            _wait_gather(r, 0, 0)
        else:
            @pl.when(n + 1 < nt)
            def _():
                _wait_idx(j1, r1)
                _gather(r1, 0, 0)
            _wait_gather(r, 1, 1)

        @pl.when(u >= 2)
        def _():
            _wait_scatter(h)

        if h == 0:
            _wait_ea(s + n * 16, r)

        @plsc.parallel_loop(0, _H // _C)
        def _idx_copy(i):
            idx[h][pl.ds(i * _C, _C)] = src[r][h, pl.ds(i * _C, _C)]

        gv, e0v, e1v, ov = g[h], ea0[r], ea1[r], o[h]

        @plsc.parallel_loop(0, _H, unroll=2)
        def _edge(e):
            g0 = gv[e, pl.ds(0, 16)]
            g1x = gv[e, pl.ds(16, 16)]
            g1y = gv[e, pl.ds(32, 16)]
            g1z = gv[e, pl.ds(48, 16)]
            es = jnp.full((_C,), e + h * _H, jnp.int32)
            a0 = plsc.load_gather(e0v, [ccol + es])
            a1x = plsc.load_gather(e1v, [a1c[0] + es])
            a1y = plsc.load_gather(e1v, [a1c[1] + es])
            a1z = plsc.load_gather(e1v, [a1c[2] + es])
            o0 = g0 * a0 + g1x * a1x + g1y * a1y + g1z * a1z
            px = g0 * a1x + g1x * a0
            py = g0 * a1y + g1y * a0
            pz = g0 * a1z + g1z * a0
            ov[e, pl.ds(0, 16)] = jnp.where(cm, o0, py)
            ov[e, pl.ds(16, 16)] = jnp.where(cm, px, pz)

        pltpu.async_copy(ov, acc.at[idx[h]], semsc[h], add=True)

    def outer(m, _):
        for k in range(4):
            u = m * 4 + k
            body(u, k % 2, (k // 2) % 2)
        return 0

    lax.fori_loop(0, 2 * _MAIN // 4, outer, 0)

    @pl.when(s < _EXTRA)
    def _():
        body(2 * _MAIN, 0, 0)
        body(2 * _MAIN + 1, 1, 0)

    _wait_scatter(0)
    _wait_scatter(1)
    plsc.subcore_barrier()

    pltpu.sync_copy(acc.at[pl.ds(row0, _RPT)],
                    out_hbm.at[pl.ds(c * _ACC_ROWS + row0, _RPT)])


def kernel(h_0, h_1, rel_pos, edge_index, edge_attr_0, edge_attr_1,
           channel_weights):
    del rel_pos, channel_weights
    n = h_0.shape[0]
    nb = _NCHUNK
    htab = jnp.concatenate(
        [h_0, h_1[:, :, 0], h_1[:, :, 1], h_1[:, :, 2]], axis=1)
    src = edge_index[0].astype(jnp.int32)
    nbr2 = edge_index[1].astype(jnp.int32)
    ea0_t = (edge_attr_0.T.reshape(2, 8, nb, 128)
             .transpose(0, 2, 1, 3).reshape(-1))
    ea1_t = (edge_attr_1.transpose(2, 1, 0).reshape(3, 2, 8, nb, 128)
             .transpose(0, 1, 3, 2, 4).reshape(-1))
    out = _edge_kernel(htab, src, nbr2, ea0_t,
                       ea1_t).reshape(2, _ACC_ROWS, 32)
    a, b = out[0, :n], out[1, :n]
    out0 = a[:, :16]
    out1 = jnp.stack([a[:, 16:], b[:, :16], b[:, 16:]], axis=-1)
    return (out0, out1)


# speedup vs baseline: 2.2106x; 2.1841x over previous
import functools

import jax
import jax.numpy as jnp
from jax import lax
from jax.experimental import pallas as pl
from jax.experimental.pallas import tpu as pltpu
from jax.experimental.pallas import tpu_sc as plsc

_N = 50000
_E = 800000
_C = 16
_K = 128
_H = 64
_NCHUNK = _E // _K
_MAIN = 390
_EXTRA = _NCHUNK - 16 * _MAIN
_ACC_ROWS = 50048
_RPT = _ACC_ROWS // 16

_mesh = plsc.VectorSubcoreMesh(core_axis_name="c", subcore_axis_name="s")


@functools.partial(
    pl.kernel,
    out_type=jax.ShapeDtypeStruct((2 * _ACC_ROWS, 32), jnp.float32),
    mesh=_mesh,
    compiler_params=pltpu.CompilerParams(needs_layout_passes=False,
                                         use_tc_tiling_on_sc=False),
    scratch_types=[
        [pltpu.VMEM((_K,), jnp.int32)] * 2,
        [pltpu.VMEM((2, _H), jnp.int32)] * 2,
        [pltpu.VMEM((16, 136), jnp.float32)] * 2,
        [pltpu.VMEM((48, 136), jnp.float32)] * 2,
        [pltpu.VMEM((_H, 64), jnp.float32)] * 2,
        [pltpu.VMEM((_H, 32), jnp.float32)] * 2,
        [pltpu.VMEM((_H,), jnp.int32)] * 2,
        pltpu.VMEM_SHARED((_ACC_ROWS, 32), jnp.float32),
        [pltpu.SemaphoreType.DMA] * 2,
        [pltpu.SemaphoreType.DMA] * 2,
        [pltpu.SemaphoreType.DMA] * 2,
        [pltpu.SemaphoreType.DMA] * 2,
    ],
)
def _edge_kernel(htab, srcm, nbrm, ea0m, ea1m, out_hbm,
                 nbr, src, ea0, ea1, g, o, idx, acc,
                 semidx, semea, semg, semsc):
    c = lax.axis_index("c")
    s = lax.axis_index("s")
    nt = jnp.where(s < _EXTRA, _MAIN + 1, _MAIN)

    cm = (jnp.zeros((_C,), jnp.int32) + c) == 0
    zeros16 = jnp.zeros((_C,), jnp.float32)
    iota = lax.iota(jnp.int32, _C)
    arow = [iota, iota + 16, iota + 32]

    def _idx_refs(j, r):
        return (
            (nbrm.at[pl.ds(j * _K, _K)], nbr[r].at[pl.ds(0, _K)]),
            (srcm.at[pl.ds(j * _K, _H)], src[r].at[0]),
            (srcm.at[pl.ds(j * _K + _H, _H)], src[r].at[1]),
        )

    def _ea_refs(j, r):
        refs = []
        for q in range(2):
            refs.append((ea0m.at[pl.ds((q * _NCHUNK + j) * 8, 8)],
                         ea0[r].at[pl.ds(q * 8, 8), pl.ds(0, 128)]))
        for q in range(6):
            refs.append((ea1m.at[pl.ds((q * _NCHUNK + j) * 8, 8)],
                         ea1[r].at[pl.ds(q * 8, 8), pl.ds(0, 128)]))
        return refs

    def _issue_lin(j, r):
        for a, b in _idx_refs(j, r):
            pltpu.async_copy(a, b, semidx[r])
        for a, b in _ea_refs(j, r):
            pltpu.async_copy(a, b, semea[r])

    def _wait_idx(j, r):
        for a, b in _idx_refs(j, r):
            pltpu.make_async_copy(a, b, semidx[r]).wait()

    def _wait_ea(j, r):
        for a, b in _ea_refs(j, r):
            pltpu.make_async_copy(a, b, semea[r]).wait()

    def _gather(r, h, p):
        pltpu.async_copy(htab.at[nbr[r].at[pl.ds(h * _H, _H)]], g[p], semg[p])

    def _wait_gather(r, h, p):
        pltpu.make_async_copy(htab.at[nbr[r].at[pl.ds(h * _H, _H)]],
                              g[p], semg[p]).wait()

    def _wait_scatter(h):
        pltpu.make_async_copy(o[h], acc.at[idx[h]], semsc[h]).wait()

    _issue_lin(s, 0)

    @plsc.parallel_loop(0, _H)
    def _zero_row(e):
        o[0][e, pl.ds(0, 16)] = zeros16
        o[0][e, pl.ds(16, 16)] = zeros16

    row0 = s * _RPT
    for k in range(_RPT // _H):
        pltpu.sync_copy(o[0], acc.at[pl.ds(row0 + k * _H, _H)])
    pltpu.sync_copy(o[0].at[pl.ds(0, _RPT % _H)],
                    acc.at[pl.ds(row0 + _RPT // _H * _H, _RPT % _H)])

    _wait_idx(s, 0)
    _gather(0, 0, 0)
    plsc.subcore_barrier()

    def body(u, h, r):
        n = u // 2
        r1 = 1 - r
        j1 = s + (n + 1) * 16

        if h == 0:
            @pl.when(n + 1 < nt)
            def _():
                _issue_lin(j1, r1)
            _gather(r, 1, 1)
            _wait_gather(r, 0, 0)
        else:
            @pl.when(n + 1 < nt)
            def _():
                _wait_idx(j1, r1)
                _gather(r1, 0, 0)
            _wait_gather(r, 1, 1)

        @pl.when(u >= 2)
        def _():
            _wait_scatter(h)

        if h == 0:
            _wait_ea(s + n * 16, r)

        @plsc.parallel_loop(0, _H // _C)
        def _idx_copy(i):
            idx[h][pl.ds(i * _C, _C)] = src[r][h, pl.ds(i * _C, _C)]

        gv, e0v, e1v, ov = g[h], ea0[r], ea1[r], o[h]

        @plsc.parallel_loop(0, _H, unroll=2)
        def _edge(e):
            g0 = gv[e, pl.ds(0, 16)]
            g1x = gv[e, pl.ds(16, 16)]
            g1y = gv[e, pl.ds(32, 16)]
            g1z = gv[e, pl.ds(48, 16)]
            es = jnp.full((_C,), e + h * _H, jnp.int32)
            a0 = plsc.load_gather(e0v, [arow[0], es])
            a1x = plsc.load_gather(e1v, [arow[0], es])
            a1y = plsc.load_gather(e1v, [arow[1], es])
            a1z = plsc.load_gather(e1v, [arow[2], es])
            o0 = g0 * a0 + g1x * a1x + g1y * a1y + g1z * a1z
            px = g0 * a1x + g1x * a0
            py = g0 * a1y + g1y * a0
            pz = g0 * a1z + g1z * a0
            ov[e, pl.ds(0, 16)] = jnp.where(cm, o0, py)
            ov[e, pl.ds(16, 16)] = jnp.where(cm, px, pz)

        pltpu.async_copy(ov, acc.at[idx[h]], semsc[h], add=True)

    def outer(m, _):
        for k in range(4):
            u = m * 4 + k
            body(u, k % 2, (k // 2) % 2)
        return 0

    lax.fori_loop(0, 2 * _MAIN // 4, outer, 0)

    @pl.when(s < _EXTRA)
    def _():
        body(2 * _MAIN, 0, 0)
        body(2 * _MAIN + 1, 1, 0)

    _wait_scatter(0)
    _wait_scatter(1)
    plsc.subcore_barrier()

    pltpu.sync_copy(acc.at[pl.ds(row0, _RPT)],
                    out_hbm.at[pl.ds(c * _ACC_ROWS + row0, _RPT)])


def kernel(h_0, h_1, rel_pos, edge_index, edge_attr_0, edge_attr_1,
           channel_weights):
    del rel_pos, channel_weights
    n = h_0.shape[0]
    nb = _NCHUNK
    htab = jnp.concatenate(
        [h_0, h_1[:, :, 0], h_1[:, :, 1], h_1[:, :, 2],
         jnp.zeros((n, 64), jnp.float32)], axis=1).reshape(2 * n, 64)
    src = edge_index[0].astype(jnp.int32)
    nbr2 = edge_index[1].astype(jnp.int32) * 2
    ea0_t = (edge_attr_0.T.reshape(2, 8, nb, 128)
             .transpose(0, 2, 1, 3).reshape(-1, 128))
    ea1_t = (edge_attr_1.transpose(2, 1, 0).reshape(3, 2, 8, nb, 128)
             .transpose(0, 1, 3, 2, 4).reshape(-1, 128))
    out = _edge_kernel(htab, src, nbr2, ea0_t,
                       ea1_t).reshape(2, _ACC_ROWS, 32)
    a, b = out[0, :n], out[1, :n]
    out0 = a[:, :16]
    out1 = jnp.stack([a[:, 16:], b[:, :16], b[:, 16:]], axis=-1)
    return (out0, out1)
